# Initial kernel scaffold; baseline (speedup 1.0000x reference)
import jax
import jax.numpy as jnp
from jax.experimental import pallas as pl


def kernel(x, codebooks, bias, ln_weight, centroids, codes, indices):
    raise NotImplementedError("write your pallas kernel here")


# speedup vs baseline: 7.5381x; 7.5381x over previous
import jax
import jax.numpy as jnp
from jax.experimental import pallas as pl
from jax.experimental.pallas import tpu as pltpu

IN_F = 2048
OUT_F = 4096
NCB = 16
NCODES = 256
DSUB = IN_F // NCB
NCL = 32
EPS = 1e-5

RB = 256
CB = 512


def _routing_body(x_ref, cb_ref, ln_ref, cent_ref, clcol_ref, mask_ref,
                  centw_ref, sel_ref):
    r = pl.program_id(0)

    @pl.when(r == 0)
    def _init():
        cb = cb_ref[...]
        cents = cent_ref[...]
        iota = jax.lax.broadcasted_iota(jnp.int32, (NCL, NCODES), 1)
        for c in range(NCB):
            oh = (cents[c][:, None] == iota).astype(jnp.float32)
            centw_ref[:, c * DSUB:(c + 1) * DSUB] = jnp.dot(
                oh, cb[c], preferred_element_type=jnp.float32)
        sel_ref[...] = jnp.zeros_like(sel_ref)

    x = x_ref[...]
    centw = centw_ref[...]
    dots = jax.lax.dot_general(x, centw, (((1,), (1,)), ((), ())),
                               preferred_element_type=jnp.float32)
    mean = jnp.mean(dots, axis=1, keepdims=True)
    var = jnp.mean((dots - mean) ** 2, axis=1, keepdims=True)
    s = (dots - mean) * jax.lax.rsqrt(var + EPS) * ln_ref[...]
    rowmax = jnp.max(s, axis=1, keepdims=True)
    colidx = jax.lax.broadcasted_iota(jnp.int32, s.shape, 1)
    am = jnp.min(jnp.where(s >= rowmax, colidx, NCL), axis=1)
    hit = (am[:, None] == jax.lax.broadcasted_iota(jnp.int32, (RB, NCL), 1))
    sel_ref[0, :] = jnp.maximum(sel_ref[0, :], jnp.max(hit.astype(jnp.float32), axis=0))

    @pl.when(r == pl.num_programs(0) - 1)
    def _fin():
        sel = sel_ref[...]
        clc = clcol_ref[...]
        eq = (clc == jax.lax.broadcasted_iota(jnp.int32, (NCL, OUT_F), 0)
              ).astype(jnp.float32)
        mask_ref[...] = jnp.dot(sel, eq, preferred_element_type=jnp.float32)


def _main_body(codes_ref, x_ref, bias_ref, mask_ref, cb_ref, out_ref, w_ref):
    r = pl.program_id(1)

    @pl.when(r == 0)
    def _build_w():
        cb = cb_ref[...]
        codes = codes_ref[...]
        iota = jax.lax.broadcasted_iota(jnp.int32, (CB, NCODES), 1)
        for k in range(NCB):
            oh = (codes[k][:, None] == iota).astype(jnp.float32)
            w_ref[:, k * DSUB:(k + 1) * DSUB] = jnp.dot(
                oh, cb[k], preferred_element_type=jnp.float32)

    x = x_ref[...]
    w = w_ref[...]
    y = jax.lax.dot_general(x, w, (((1,), (1,)), ((), ())),
                            preferred_element_type=jnp.float32)
    out_ref[...] = (y + bias_ref[...]) * mask_ref[...]


def kernel(x, codebooks, bias, ln_weight, centroids, codes, indices):
    shape = x.shape
    xin = x.reshape(-1, shape[-1])
    n = xin.shape[0]
    ncl, csz = indices.shape
    cluster_of_col = jnp.zeros((OUT_F,), jnp.int32).at[indices.reshape(-1)].set(
        jnp.repeat(jnp.arange(ncl, dtype=jnp.int32), csz))
    clc2d = cluster_of_col.reshape(1, OUT_F)
    ln2d = ln_weight.reshape(1, NCL)
    bias2d = bias.reshape(1, OUT_F)

    mask = pl.pallas_call(
        _routing_body,
        grid=(n // RB,),
        in_specs=[
            pl.BlockSpec((RB, IN_F), lambda r: (r, 0)),
            pl.BlockSpec((NCB, NCODES, DSUB), lambda r: (0, 0, 0)),
            pl.BlockSpec((1, NCL), lambda r: (0, 0)),
            pl.BlockSpec((NCB, NCL), lambda r: (0, 0)),
            pl.BlockSpec((1, OUT_F), lambda r: (0, 0)),
        ],
        out_specs=pl.BlockSpec((1, OUT_F), lambda r: (0, 0)),
        out_shape=jax.ShapeDtypeStruct((1, OUT_F), jnp.float32),
        scratch_shapes=[
            pltpu.VMEM((NCL, IN_F), jnp.float32),
            pltpu.VMEM((1, NCL), jnp.float32),
        ],
    )(xin, codebooks, ln2d, centroids, clc2d)

    y = pl.pallas_call(
        _main_body,
        grid=(OUT_F // CB, n // RB),
        in_specs=[
            pl.BlockSpec((NCB, CB), lambda c, r: (0, c)),
            pl.BlockSpec((RB, IN_F), lambda c, r: (r, 0)),
            pl.BlockSpec((1, CB), lambda c, r: (0, c)),
            pl.BlockSpec((1, CB), lambda c, r: (0, c)),
            pl.BlockSpec((NCB, NCODES, DSUB), lambda c, r: (0, 0, 0)),
        ],
        out_specs=pl.BlockSpec((RB, CB), lambda c, r: (r, c)),
        out_shape=jax.ShapeDtypeStruct((n, OUT_F), jnp.float32),
        scratch_shapes=[pltpu.VMEM((CB, IN_F), jnp.float32)],
    )(codes, xin, bias2d, mask, codebooks)

    return y.reshape(*shape[:-1], OUT_F)


# speedup vs baseline: 7.6941x; 1.0207x over previous
import jax
import jax.numpy as jnp
from jax.experimental import pallas as pl
from jax.experimental.pallas import tpu as pltpu

IN_F = 2048
OUT_F = 4096
NCB = 16
NCODES = 256
DSUB = IN_F // NCB
NCL = 32
EPS = 1e-5

RB = 256
CB = 512


def _routing_body(x_ref, cb_ref, ln_ref, cent_ref, clcol_ref, mask_ref,
                  centw_ref, sel_ref):
    r = pl.program_id(0)

    @pl.when(r == 0)
    def _init():
        cb = cb_ref[...]
        cents = cent_ref[...]
        iota = jax.lax.broadcasted_iota(jnp.int32, (NCL, NCODES), 1)
        for c in range(NCB):
            oh = (cents[c][:, None] == iota).astype(jnp.float32)
            centw_ref[:, c * DSUB:(c + 1) * DSUB] = jnp.dot(
                oh, cb[c], preferred_element_type=jnp.float32)
        sel_ref[...] = jnp.zeros_like(sel_ref)

    x = x_ref[...]
    centw = centw_ref[...]
    dots = jax.lax.dot_general(x, centw, (((1,), (1,)), ((), ())),
                               preferred_element_type=jnp.float32)
    mean = jnp.mean(dots, axis=1, keepdims=True)
    var = jnp.mean((dots - mean) ** 2, axis=1, keepdims=True)
    s = (dots - mean) * jax.lax.rsqrt(var + EPS) * ln_ref[...]
    rowmax = jnp.max(s, axis=1, keepdims=True)
    colidx = jax.lax.broadcasted_iota(jnp.int32, s.shape, 1)
    am = jnp.min(jnp.where(s >= rowmax, colidx, NCL), axis=1)
    hit = (am[:, None] == jax.lax.broadcasted_iota(jnp.int32, (RB, NCL), 1))
    sel_ref[0, :] = jnp.maximum(sel_ref[0, :], jnp.max(hit.astype(jnp.float32), axis=0))

    @pl.when(r == pl.num_programs(0) - 1)
    def _fin():
        sel = sel_ref[...]
        clc = clcol_ref[...]
        eq = (clc == jax.lax.broadcasted_iota(jnp.int32, (NCL, OUT_F), 0)
              ).astype(jnp.float32)
        mask_ref[...] = jnp.dot(sel, eq, preferred_element_type=jnp.float32)


def _main_body(codes_ref, x_ref, bias_ref, mask_ref, cb_ref, out_ref, w_ref):
    r = pl.program_id(1)

    @pl.when(r == 0)
    def _build_w():
        cb = cb_ref[...]
        codes = codes_ref[...]
        iota = jax.lax.broadcasted_iota(jnp.int32, (CB, NCODES), 1)
        for k in range(NCB):
            oh = (codes[k][:, None] == iota).astype(jnp.float32)
            w_ref[:, k * DSUB:(k + 1) * DSUB] = jnp.dot(
                oh, cb[k], preferred_element_type=jnp.float32).astype(jnp.bfloat16)

    x = x_ref[...]
    w = w_ref[...]
    y = jax.lax.dot_general(x, w, (((1,), (1,)), ((), ())),
                            preferred_element_type=jnp.float32)
    out_ref[...] = (y + bias_ref[...]) * mask_ref[...]


def kernel(x, codebooks, bias, ln_weight, centroids, codes, indices):
    shape = x.shape
    xin = x.reshape(-1, shape[-1])
    n = xin.shape[0]
    ncl, csz = indices.shape
    cluster_of_col = jnp.zeros((OUT_F,), jnp.int32).at[indices.reshape(-1)].set(
        jnp.repeat(jnp.arange(ncl, dtype=jnp.int32), csz))
    clc2d = cluster_of_col.reshape(1, OUT_F)
    ln2d = ln_weight.reshape(1, NCL)
    bias2d = bias.reshape(1, OUT_F)

    mask = pl.pallas_call(
        _routing_body,
        grid=(n // RB,),
        in_specs=[
            pl.BlockSpec((RB, IN_F), lambda r: (r, 0)),
            pl.BlockSpec((NCB, NCODES, DSUB), lambda r: (0, 0, 0)),
            pl.BlockSpec((1, NCL), lambda r: (0, 0)),
            pl.BlockSpec((NCB, NCL), lambda r: (0, 0)),
            pl.BlockSpec((1, OUT_F), lambda r: (0, 0)),
        ],
        out_specs=pl.BlockSpec((1, OUT_F), lambda r: (0, 0)),
        out_shape=jax.ShapeDtypeStruct((1, OUT_F), jnp.float32),
        scratch_shapes=[
            pltpu.VMEM((NCL, IN_F), jnp.float32),
            pltpu.VMEM((1, NCL), jnp.float32),
        ],
    )(xin, codebooks, ln2d, centroids, clc2d)

    y = pl.pallas_call(
        _main_body,
        grid=(OUT_F // CB, n // RB),
        in_specs=[
            pl.BlockSpec((NCB, CB), lambda c, r: (0, c)),
            pl.BlockSpec((RB, IN_F), lambda c, r: (r, 0)),
            pl.BlockSpec((1, CB), lambda c, r: (0, c)),
            pl.BlockSpec((1, CB), lambda c, r: (0, c)),
            pl.BlockSpec((NCB, NCODES, DSUB), lambda c, r: (0, 0, 0)),
        ],
        out_specs=pl.BlockSpec((RB, CB), lambda c, r: (r, c)),
        out_shape=jax.ShapeDtypeStruct((n, OUT_F), jnp.float32),
        scratch_shapes=[pltpu.VMEM((CB, IN_F), jnp.bfloat16)],
    )(codes, xin.astype(jnp.bfloat16), bias2d, mask, codebooks)

    return y.reshape(*shape[:-1], OUT_F)


# speedup vs baseline: 11.2442x; 1.4614x over previous
import jax
import jax.numpy as jnp
from jax.experimental import pallas as pl
from jax.experimental.pallas import tpu as pltpu

IN_F = 2048
OUT_F = 4096
NCB = 16
NCODES = 256
DSUB = IN_F // NCB
NCL = 32
EPS = 1e-5

RB = 256
CB = 512


def _routing_body(x_ref, cb_ref, ln_ref, cent_ref, clcol_ref, mask_ref,
                  centw_ref, sel_ref):
    r = pl.program_id(0)

    @pl.when(r == 0)
    def _init():
        cb = cb_ref[...]
        cents = cent_ref[...]
        iota = jax.lax.broadcasted_iota(jnp.int32, (NCL, NCODES), 1)
        for c in range(NCB):
            oh = (cents[c][:, None] == iota).astype(jnp.bfloat16)
            centw_ref[:, c * DSUB:(c + 1) * DSUB] = jnp.dot(
                oh, cb[c], preferred_element_type=jnp.float32).astype(jnp.bfloat16)
        sel_ref[...] = jnp.zeros_like(sel_ref)

    x = x_ref[...]
    centw = centw_ref[...]
    dots = jax.lax.dot_general(x, centw, (((1,), (1,)), ((), ())),
                               preferred_element_type=jnp.float32)
    mean = jnp.mean(dots, axis=1, keepdims=True)
    var = jnp.mean((dots - mean) ** 2, axis=1, keepdims=True)
    s = (dots - mean) * jax.lax.rsqrt(var + EPS) * ln_ref[...]
    rowmax = jnp.max(s, axis=1, keepdims=True)
    colidx = jax.lax.broadcasted_iota(jnp.int32, s.shape, 1)
    am = jnp.min(jnp.where(s >= rowmax, colidx, NCL), axis=1)
    hit = (am[:, None] == jax.lax.broadcasted_iota(jnp.int32, (RB, NCL), 1))
    sel_ref[0, :] = jnp.maximum(sel_ref[0, :], jnp.max(hit.astype(jnp.float32), axis=0))

    @pl.when(r == pl.num_programs(0) - 1)
    def _fin():
        sel = sel_ref[...]
        clc = clcol_ref[...]
        eq = (clc == jax.lax.broadcasted_iota(jnp.int32, (NCL, OUT_F), 0)
              ).astype(jnp.float32)
        mask_ref[...] = jnp.dot(sel, eq, preferred_element_type=jnp.float32)


def _main_body(codes_ref, x_ref, bias_ref, mask_ref, cb_ref, out_ref, w_ref):
    r = pl.program_id(0)

    @pl.when(r == 0)
    def _build_w():
        cb = cb_ref[...]
        iota = jax.lax.broadcasted_iota(jnp.int32, (CB, NCODES), 1)
        for j in range(OUT_F // CB):
            codes = codes_ref[:, j * CB:(j + 1) * CB]
            for k in range(NCB):
                oh = (codes[k][:, None] == iota).astype(jnp.bfloat16)
                w_ref[j * CB:(j + 1) * CB, k * DSUB:(k + 1) * DSUB] = jnp.dot(
                    oh, cb[k], preferred_element_type=jnp.float32).astype(jnp.bfloat16)

    x = x_ref[...]
    w = w_ref[...]
    y = jax.lax.dot_general(x, w, (((1,), (1,)), ((), ())),
                            preferred_element_type=jnp.float32)
    out_ref[...] = (y + bias_ref[...]) * mask_ref[...]


def kernel(x, codebooks, bias, ln_weight, centroids, codes, indices):
    shape = x.shape
    xin = x.reshape(-1, shape[-1])
    n = xin.shape[0]
    ncl, csz = indices.shape
    xbf = xin.astype(jnp.bfloat16)
    cluster_of_col = jnp.zeros((OUT_F,), jnp.int32).at[indices.reshape(-1)].set(
        jnp.repeat(jnp.arange(ncl, dtype=jnp.int32), csz))
    clc2d = cluster_of_col.reshape(1, OUT_F)
    ln2d = ln_weight.reshape(1, NCL)
    bias2d = bias.reshape(1, OUT_F)

    mask = pl.pallas_call(
        _routing_body,
        grid=(n // RB,),
        in_specs=[
            pl.BlockSpec((RB, IN_F), lambda r: (r, 0)),
            pl.BlockSpec((NCB, NCODES, DSUB), lambda r: (0, 0, 0)),
            pl.BlockSpec((1, NCL), lambda r: (0, 0)),
            pl.BlockSpec((NCB, NCL), lambda r: (0, 0)),
            pl.BlockSpec((1, OUT_F), lambda r: (0, 0)),
        ],
        out_specs=pl.BlockSpec((1, OUT_F), lambda r: (0, 0)),
        out_shape=jax.ShapeDtypeStruct((1, OUT_F), jnp.float32),
        scratch_shapes=[
            pltpu.VMEM((NCL, IN_F), jnp.bfloat16),
            pltpu.VMEM((1, NCL), jnp.float32),
        ],
    )(xbf, codebooks, ln2d, centroids, clc2d)

    y = pl.pallas_call(
        _main_body,
        grid=(n // RB,),
        in_specs=[
            pl.BlockSpec((NCB, OUT_F), lambda r: (0, 0)),
            pl.BlockSpec((RB, IN_F), lambda r: (r, 0)),
            pl.BlockSpec((1, OUT_F), lambda r: (0, 0)),
            pl.BlockSpec((1, OUT_F), lambda r: (0, 0)),
            pl.BlockSpec((NCB, NCODES, DSUB), lambda r: (0, 0, 0)),
        ],
        out_specs=pl.BlockSpec((RB, OUT_F), lambda r: (r, 0)),
        out_shape=jax.ShapeDtypeStruct((n, OUT_F), jnp.float32),
        scratch_shapes=[pltpu.VMEM((OUT_F, IN_F), jnp.bfloat16)],
    )(codes, xbf, bias2d, mask, codebooks)

    return y.reshape(*shape[:-1], OUT_F)
